# Initial kernel scaffold; baseline (speedup 1.0000x reference)
import jax
import jax.numpy as jnp
from jax.experimental import pallas as pl


def kernel(x, tape_re, tape_im, eta, torque_bias_re, torque_bias_im):
    raise NotImplementedError("write your pallas kernel here")


# speedup vs baseline: 25.5783x; 25.5783x over previous
import functools

import jax
import jax.numpy as jnp
from jax.experimental import pallas as pl
from jax.experimental.pallas import tpu as pltpu

_DIM = 2048
_B = 8
_TOPK = 8
_MAXT = 16
_TSLOTS = 128
_BETA = 0.05
_GAMMA = 0.9
_LIFE = 5
_PAIRS = [(i, j) for i in range(_TOPK) for j in range(i + 1, _TOPK)]
_NPAIR = len(_PAIRS)
_NEG = -1.0e30


def _step_kernel(x_ref, tre_ref, tim_ref, eta_ref, tbre_ref, tbim_ref,
                 out_ref,
                 sre_ref, sim_ref, ti_ref, tj_ref, tmre_ref, tmim_ref,
                 tcnt_ref):
    t = pl.program_id(0)

    @pl.when(t == 0)
    def _init():
        tr = tre_ref[0:1, :]
        tm = tim_ref[0:1, :]
        nrm = jnp.sqrt(jnp.sum(tr * tr + tm * tm))
        nrm = jnp.maximum(nrm, 1e-8)
        sre_ref[...] = jnp.broadcast_to(tr / nrm, (_B, _DIM))
        sim_ref[...] = jnp.broadcast_to(tm / nrm, (_B, _DIM))
        ti_ref[...] = jnp.zeros((_B, _TSLOTS), jnp.int32)
        tj_ref[...] = jnp.zeros((_B, _TSLOTS), jnp.int32)
        tmre_ref[...] = jnp.zeros((_B, _TSLOTS), jnp.float32)
        tmim_ref[...] = jnp.zeros((_B, _TSLOTS), jnp.float32)
        tcnt_ref[...] = jnp.zeros((_B, _TSLOTS), jnp.int32)

    eta = jnp.abs(eta_ref[0, 0])
    s_re = sre_ref[...]
    s_im = sim_ref[...]
    h = x_ref[0, :, :]

    c_re = h * s_re
    c_im = h * s_im
    mag = jnp.sqrt(c_re * c_re + c_im * c_im)

    iota_d = jax.lax.broadcasted_iota(jnp.int32, (_B, _DIM), 1)

    work = mag
    top_mag = []
    top_cre = []
    top_cim = []
    top_idx = []
    top_sre = []
    top_sim = []
    for _ in range(_TOPK):
        m = jnp.max(work, axis=1, keepdims=True)
        idx = jnp.min(jnp.where(work == m, iota_d, _DIM), axis=1,
                      keepdims=True)
        oh = iota_d == idx
        ohf = oh.astype(jnp.float32)
        top_mag.append(m)
        top_idx.append(idx)
        top_cre.append(jnp.sum(ohf * c_re, axis=1, keepdims=True))
        top_cim.append(jnp.sum(ohf * c_im, axis=1, keepdims=True))
        top_sre.append(jnp.sum(ohf * s_re, axis=1, keepdims=True))
        top_sim.append(jnp.sum(ohf * s_im, axis=1, keepdims=True))
        work = jnp.where(oh, -1.0, work)

    score_cols = []
    st_re_cols = []
    st_im_cols = []
    for (i, j) in _PAIRS:
        score_cols.append(top_cre[i] * top_cre[j] + top_cim[i] * top_cim[j])
        pr_re = top_sre[i] * top_sre[j] - top_sim[i] * top_sim[j]
        pr_im = top_sre[i] * top_sim[j] + top_sim[i] * top_sre[j]
        pm = jnp.maximum(jnp.sqrt(pr_re * pr_re + pr_im * pr_im), 1e-8)
        st_re_cols.append(_BETA * pr_re / pm)
        st_im_cols.append(_BETA * pr_im / pm)
    scores = jnp.concatenate(score_cols, axis=1)

    pos = scores > 0.0
    npos = jnp.sum(pos.astype(jnp.int32), axis=1, keepdims=True)
    n_to_bind = (1 + (npos >= 14).astype(jnp.int32)
                 + (npos >= 20).astype(jnp.int32)
                 + (npos >= 27).astype(jnp.int32))
    theta_idx = jnp.minimum(n_to_bind - 1, jnp.maximum(npos - 1, 0))

    iota_p = jax.lax.broadcasted_iota(jnp.int32, (_B, _NPAIR), 1)
    work2 = jnp.where(pos, scores, _NEG)
    theta = jnp.full((_B, 1), _NEG, jnp.float32)
    for r in range(4):
        m2 = jnp.max(work2, axis=1, keepdims=True)
        theta = jnp.where(theta_idx == r, m2, theta)
        idx2 = jnp.min(jnp.where(work2 == m2, iota_p, _NPAIR), axis=1,
                       keepdims=True)
        work2 = jnp.where(iota_p == idx2, _NEG, work2)
    selected = (scores >= theta) & (npos > 0)

    ti = ti_ref[...]
    tj = tj_ref[...]
    tm_re = tmre_ref[...]
    tm_im = tmim_ref[...]
    tcnt = tcnt_ref[...]
    iota_s = jax.lax.broadcasted_iota(jnp.int32, (_B, _TSLOTS), 1)
    for p, (i, j) in enumerate(_PAIRS):
        sel = selected[:, p:p + 1]
        pci = top_idx[i]
        pcj = top_idx[j]
        active = tcnt > 0
        match = active & (((ti == pci) & (tj == pcj))
                          | ((ti == pcj) & (tj == pci)))
        any_match = jnp.sum(match.astype(jnp.int32), axis=1,
                            keepdims=True) > 0
        tcnt = jnp.where(match & sel, _LIFE, tcnt)
        n_active = jnp.sum(active.astype(jnp.int32), axis=1, keepdims=True)
        can_append = sel & jnp.logical_not(any_match) & (n_active < _MAXT)
        free_idx = jnp.min(jnp.where(active, _TSLOTS + 1, iota_s), axis=1,
                           keepdims=True)
        app = (iota_s == free_idx) & can_append
        ti = jnp.where(app, pci, ti)
        tj = jnp.where(app, pcj, tj)
        tm_re = jnp.where(app, st_re_cols[p], tm_re)
        tm_im = jnp.where(app, st_im_cols[p], tm_im)
        tcnt = jnp.where(app, _LIFE, tcnt)

    tm_re = tm_re * _GAMMA
    tm_im = tm_im * _GAMMA
    tcnt = tcnt - 1
    tmag = jnp.sqrt(tm_re * tm_re + tm_im * tm_im)
    survive = (tcnt > 0) & (tmag > 1e-6)
    tcnt = jnp.where(survive, tcnt, 0)

    sre_next = s_re
    sim_next = s_im
    ti_ref[...] = ti
    tj_ref[...] = tj
    tmre_ref[...] = tm_re
    tmim_ref[...] = tm_im
    tcnt_ref[...] = tcnt

    aug_re = jnp.zeros((_B, _DIM), jnp.float32)
    aug_im = jnp.zeros((_B, _DIM), jnp.float32)
    for k in range(_MAXT):
        alive = tcnt[:, k:k + 1] > 0
        cre = jnp.where(alive, 0.1 * tm_re[:, k:k + 1], 0.0)
        cim = jnp.where(alive, 0.1 * tm_im[:, k:k + 1], 0.0)
        oh = (iota_d == ti[:, k:k + 1]) | (iota_d == tj[:, k:k + 1])
        ohf = oh.astype(jnp.float32)
        aug_re = aug_re + ohf * cre
        aug_im = aug_im + ohf * cim

    ca_re = h * (sre_next + aug_re)
    ca_im = h * (sim_next + aug_im)
    abs_im = jnp.abs(ca_im)
    res_m = ((ca_re > 1e-6) & (abs_im < ca_re)).astype(jnp.float32)
    tor_m = ((ca_re < -1e-6) | (abs_im >= jnp.abs(ca_re))).astype(jnp.float32)
    nonorth = res_m + tor_m
    tb_re = tbre_ref[0:1, :]
    tb_im = tbim_ref[0:1, :]
    upd_re = eta * (ca_re * nonorth + tor_m * tb_re)
    upd_im = eta * (ca_im * nonorth + tor_m * tb_im)
    sn_re = sre_next + upd_re
    sn_im = sim_next + upd_im
    nrm = jnp.sqrt(jnp.sum(sn_re * sn_re + sn_im * sn_im, axis=1,
                           keepdims=True))
    nrm = jnp.maximum(nrm, 1e-8)
    sn_re = sn_re / nrm
    sn_im = sn_im / nrm
    sre_ref[...] = sn_re
    sim_ref[...] = sn_im
    out_ref[0, :, :] = jnp.sqrt(sn_re * sn_re + sn_im * sn_im)


@functools.partial(jax.jit, static_argnames=("interpret",))
def _run(x, tape_re, tape_im, eta, torque_bias_re, torque_bias_im,
         interpret=False):
    B, T, D = x.shape
    xt = jnp.transpose(x, (1, 0, 2))
    tre = tape_re.reshape(1, D)
    tim = tape_im.reshape(1, D)
    tbre = torque_bias_re.reshape(1, D)
    tbim = torque_bias_im.reshape(1, D)
    eta2 = jnp.asarray(eta, jnp.float32).reshape(1, 1)

    out = pl.pallas_call(
        _step_kernel,
        grid=(T,),
        in_specs=[
            pl.BlockSpec((1, B, D), lambda t: (t, 0, 0)),
            pl.BlockSpec((1, D), lambda t: (0, 0)),
            pl.BlockSpec((1, D), lambda t: (0, 0)),
            pl.BlockSpec((1, 1), lambda t: (0, 0)),
            pl.BlockSpec((1, D), lambda t: (0, 0)),
            pl.BlockSpec((1, D), lambda t: (0, 0)),
        ],
        out_specs=pl.BlockSpec((1, B, D), lambda t: (t, 0, 0)),
        out_shape=jax.ShapeDtypeStruct((T, B, D), jnp.float32),
        scratch_shapes=[
            pltpu.VMEM((_B, _DIM), jnp.float32),
            pltpu.VMEM((_B, _DIM), jnp.float32),
            pltpu.VMEM((_B, _TSLOTS), jnp.int32),
            pltpu.VMEM((_B, _TSLOTS), jnp.int32),
            pltpu.VMEM((_B, _TSLOTS), jnp.float32),
            pltpu.VMEM((_B, _TSLOTS), jnp.float32),
            pltpu.VMEM((_B, _TSLOTS), jnp.int32),
        ],
        interpret=interpret,
    )(xt, tre, tim, eta2, tbre, tbim)
    return jnp.transpose(out, (1, 0, 2))


def kernel(x, tape_re, tape_im, eta, torque_bias_re, torque_bias_im):
    return _run(x, tape_re, tape_im, eta, torque_bias_re, torque_bias_im)


# speedup vs baseline: 51.8026x; 2.0253x over previous
import functools

import jax
import jax.numpy as jnp
from jax.experimental import pallas as pl
from jax.experimental.pallas import tpu as pltpu

_DIM = 2048
_B = 8
_TOPK = 8
_MAXT = 16
_TSLOTS = 128
_BETA = 0.05
_GAMMA = 0.9
_LIFE = 5
_PAIRS = [(i, j) for i in range(_TOPK) for j in range(i + 1, _TOPK)]
_NPAIR = len(_PAIRS)
_NEG = -1.0e30


def _step_kernel(x_ref, tre_ref, tim_ref, eta_ref, tbre_ref, tbim_ref,
                 out_ref,
                 sre_ref, sim_ref, ti_ref, tj_ref, tmre_ref, tmim_ref,
                 tcnt_ref):
    t = pl.program_id(0)

    @pl.when(t == 0)
    def _init():
        tr = tre_ref[0:1, :]
        tm = tim_ref[0:1, :]
        nrm = jnp.sqrt(jnp.sum(tr * tr + tm * tm))
        nrm = jnp.maximum(nrm, 1e-8)
        sre_ref[...] = jnp.broadcast_to(tr / nrm, (_B, _DIM))
        sim_ref[...] = jnp.broadcast_to(tm / nrm, (_B, _DIM))
        ti_ref[...] = jnp.zeros((_B, _TSLOTS), jnp.int32)
        tj_ref[...] = jnp.zeros((_B, _TSLOTS), jnp.int32)
        tmre_ref[...] = jnp.zeros((_B, _TSLOTS), jnp.float32)
        tmim_ref[...] = jnp.zeros((_B, _TSLOTS), jnp.float32)
        tcnt_ref[...] = jnp.zeros((_B, _TSLOTS), jnp.int32)

    eta = jnp.abs(eta_ref[0, 0])
    s_re = sre_ref[...]
    s_im = sim_ref[...]
    h = x_ref[0, :, :]

    c_re = h * s_re
    c_im = h * s_im
    mag2 = c_re * c_re + c_im * c_im

    iota_d = jax.lax.broadcasted_iota(jnp.int32, (_B, _DIM), 1)

    work = mag2
    top_cre = []
    top_cim = []
    top_idx = []
    top_sre = []
    top_sim = []
    for _ in range(_TOPK):
        m = jnp.max(work, axis=1, keepdims=True)
        idx = jnp.min(jnp.where(work == m, iota_d, _DIM), axis=1,
                      keepdims=True)
        oh = iota_d == idx
        ohf = oh.astype(jnp.float32)
        top_idx.append(idx)
        top_cre.append(jnp.sum(ohf * c_re, axis=1, keepdims=True))
        top_cim.append(jnp.sum(ohf * c_im, axis=1, keepdims=True))
        top_sre.append(jnp.sum(ohf * s_re, axis=1, keepdims=True))
        top_sim.append(jnp.sum(ohf * s_im, axis=1, keepdims=True))
        work = jnp.where(oh, -1.0, work)

    score_cols = []
    st_re_cols = []
    st_im_cols = []
    for (i, j) in _PAIRS:
        score_cols.append(top_cre[i] * top_cre[j] + top_cim[i] * top_cim[j])
        pr_re = top_sre[i] * top_sre[j] - top_sim[i] * top_sim[j]
        pr_im = top_sre[i] * top_sim[j] + top_sim[i] * top_sre[j]
        pm = jnp.maximum(jnp.sqrt(pr_re * pr_re + pr_im * pr_im), 1e-8)
        st_re_cols.append(_BETA * pr_re / pm)
        st_im_cols.append(_BETA * pr_im / pm)
    scores = jnp.concatenate(score_cols, axis=1)

    pos = scores > 0.0
    npos = jnp.sum(pos.astype(jnp.int32), axis=1, keepdims=True)
    n_to_bind = (1 + (npos >= 14).astype(jnp.int32)
                 + (npos >= 20).astype(jnp.int32)
                 + (npos >= 27).astype(jnp.int32))
    theta_idx = jnp.minimum(n_to_bind - 1, jnp.maximum(npos - 1, 0))

    iota_p = jax.lax.broadcasted_iota(jnp.int32, (_B, _NPAIR), 1)
    work2 = jnp.where(pos, scores, _NEG)
    theta = jnp.full((_B, 1), _NEG, jnp.float32)
    for r in range(4):
        m2 = jnp.max(work2, axis=1, keepdims=True)
        theta = jnp.where(theta_idx == r, m2, theta)
        idx2 = jnp.min(jnp.where(work2 == m2, iota_p, _NPAIR), axis=1,
                       keepdims=True)
        work2 = jnp.where(iota_p == idx2, _NEG, work2)
    selected = (scores >= theta) & (npos > 0)

    ti = ti_ref[...]
    tj = tj_ref[...]
    tm_re = tmre_ref[...]
    tm_im = tmim_ref[...]
    tcnt = tcnt_ref[...]
    active0 = tcnt > 0
    active0f = active0.astype(jnp.float32)
    inact0f = 1.0 - active0f
    n_active0 = jnp.sum(active0f, axis=1, keepdims=True)

    match_sel = []
    any_match_cols = []
    for p, (i, j) in enumerate(_PAIRS):
        pci = top_idx[i]
        pcj = top_idx[j]
        match = active0 & (((ti == pci) & (tj == pcj))
                           | ((ti == pcj) & (tj == pci)))
        any_match_cols.append(jnp.max(match.astype(jnp.float32), axis=1,
                                      keepdims=True))
        match_sel.append(match & selected[:, p:p + 1])
    ms = match_sel
    while len(ms) > 1:
        nxt = [a | b for a, b in zip(ms[::2], ms[1::2])]
        if len(ms) % 2:
            nxt.append(ms[-1])
        ms = nxt
    tcnt = jnp.where(ms[0], _LIFE, tcnt)

    any_match_f = jnp.concatenate(any_match_cols, axis=1)
    app_flag = selected & (any_match_f == 0.0)
    app_flagf = app_flag.astype(jnp.float32)
    lt_p = (jax.lax.broadcasted_iota(jnp.int32, (_NPAIR, _NPAIR), 0)
            < jax.lax.broadcasted_iota(jnp.int32, (_NPAIR, _NPAIR), 1)
            ).astype(jnp.float32)
    rank = jax.lax.dot(app_flagf, lt_p,
                       preferred_element_type=jnp.float32)
    can_append = app_flag & (n_active0 + rank < float(_MAXT))
    lt_s = (jax.lax.broadcasted_iota(jnp.int32, (_TSLOTS, _TSLOTS), 0)
            < jax.lax.broadcasted_iota(jnp.int32, (_TSLOTS, _TSLOTS), 1)
            ).astype(jnp.float32)
    freerank = jax.lax.dot(inact0f, lt_s,
                           preferred_element_type=jnp.float32)
    inact0 = jnp.logical_not(active0)
    for p, (i, j) in enumerate(_PAIRS):
        app = (can_append[:, p:p + 1] & inact0
               & (freerank == rank[:, p:p + 1]))
        ti = jnp.where(app, top_idx[i], ti)
        tj = jnp.where(app, top_idx[j], tj)
        tm_re = jnp.where(app, st_re_cols[p], tm_re)
        tm_im = jnp.where(app, st_im_cols[p], tm_im)
        tcnt = jnp.where(app, _LIFE, tcnt)

    tm_re = tm_re * _GAMMA
    tm_im = tm_im * _GAMMA
    tcnt = tcnt - 1
    tmag = jnp.sqrt(tm_re * tm_re + tm_im * tm_im)
    survive = (tcnt > 0) & (tmag > 1e-6)
    tcnt = jnp.where(survive, tcnt, 0)

    sre_next = s_re
    sim_next = s_im
    ti_ref[...] = ti
    tj_ref[...] = tj
    tmre_ref[...] = tm_re
    tmim_ref[...] = tm_im
    tcnt_ref[...] = tcnt

    aug_re = jnp.zeros((_B, _DIM), jnp.float32)
    aug_im = jnp.zeros((_B, _DIM), jnp.float32)
    for k in range(_MAXT):
        alive = tcnt[:, k:k + 1] > 0
        cre = jnp.where(alive, 0.1 * tm_re[:, k:k + 1], 0.0)
        cim = jnp.where(alive, 0.1 * tm_im[:, k:k + 1], 0.0)
        oh = (iota_d == ti[:, k:k + 1]) | (iota_d == tj[:, k:k + 1])
        ohf = oh.astype(jnp.float32)
        aug_re = aug_re + ohf * cre
        aug_im = aug_im + ohf * cim

    ca_re = h * (sre_next + aug_re)
    ca_im = h * (sim_next + aug_im)
    abs_im = jnp.abs(ca_im)
    res_m = ((ca_re > 1e-6) & (abs_im < ca_re)).astype(jnp.float32)
    tor_m = ((ca_re < -1e-6) | (abs_im >= jnp.abs(ca_re))).astype(jnp.float32)
    nonorth = res_m + tor_m
    tb_re = tbre_ref[0:1, :]
    tb_im = tbim_ref[0:1, :]
    upd_re = eta * (ca_re * nonorth + tor_m * tb_re)
    upd_im = eta * (ca_im * nonorth + tor_m * tb_im)
    sn_re = sre_next + upd_re
    sn_im = sim_next + upd_im
    nrm = jnp.sqrt(jnp.sum(sn_re * sn_re + sn_im * sn_im, axis=1,
                           keepdims=True))
    nrm = jnp.maximum(nrm, 1e-8)
    sn_re = sn_re / nrm
    sn_im = sn_im / nrm
    sre_ref[...] = sn_re
    sim_ref[...] = sn_im
    out_ref[0, :, :] = jnp.sqrt(sn_re * sn_re + sn_im * sn_im)


@functools.partial(jax.jit, static_argnames=("interpret",))
def _run(x, tape_re, tape_im, eta, torque_bias_re, torque_bias_im,
         interpret=False):
    B, T, D = x.shape
    xt = jnp.transpose(x, (1, 0, 2))
    tre = tape_re.reshape(1, D)
    tim = tape_im.reshape(1, D)
    tbre = torque_bias_re.reshape(1, D)
    tbim = torque_bias_im.reshape(1, D)
    eta2 = jnp.asarray(eta, jnp.float32).reshape(1, 1)

    out = pl.pallas_call(
        _step_kernel,
        grid=(T,),
        in_specs=[
            pl.BlockSpec((1, B, D), lambda t: (t, 0, 0)),
            pl.BlockSpec((1, D), lambda t: (0, 0)),
            pl.BlockSpec((1, D), lambda t: (0, 0)),
            pl.BlockSpec((1, 1), lambda t: (0, 0)),
            pl.BlockSpec((1, D), lambda t: (0, 0)),
            pl.BlockSpec((1, D), lambda t: (0, 0)),
        ],
        out_specs=pl.BlockSpec((1, B, D), lambda t: (t, 0, 0)),
        out_shape=jax.ShapeDtypeStruct((T, B, D), jnp.float32),
        scratch_shapes=[
            pltpu.VMEM((_B, _DIM), jnp.float32),
            pltpu.VMEM((_B, _DIM), jnp.float32),
            pltpu.VMEM((_B, _TSLOTS), jnp.int32),
            pltpu.VMEM((_B, _TSLOTS), jnp.int32),
            pltpu.VMEM((_B, _TSLOTS), jnp.float32),
            pltpu.VMEM((_B, _TSLOTS), jnp.float32),
            pltpu.VMEM((_B, _TSLOTS), jnp.int32),
        ],
        interpret=interpret,
    )(xt, tre, tim, eta2, tbre, tbim)
    return jnp.transpose(out, (1, 0, 2))


def kernel(x, tape_re, tape_im, eta, torque_bias_re, torque_bias_im):
    return _run(x, tape_re, tape_im, eta, torque_bias_re, torque_bias_im)


# speedup vs baseline: 67.2883x; 1.2989x over previous
import functools

import jax
import jax.numpy as jnp
from jax.experimental import pallas as pl
from jax.experimental.pallas import tpu as pltpu

_DIM = 2048
_B = 8
_TOPK = 8
_MAXT = 16
_TSLOTS = 128
_BETA = 0.05
_GAMMA = 0.9
_LIFE = 5
_PAIRS = [(i, j) for i in range(_TOPK) for j in range(i + 1, _TOPK)]
_NPAIR = len(_PAIRS)
_NEG = -1.0e30


def _step_kernel(x_ref, tre_ref, tim_ref, eta_ref, tbre_ref, tbim_ref,
                 out_ref,
                 sre_ref, sim_ref, ti_ref, tj_ref, tmre_ref, tmim_ref,
                 tcnt_ref):
    t = pl.program_id(0)

    @pl.when(t == 0)
    def _init():
        tr = tre_ref[0:1, :]
        tm = tim_ref[0:1, :]
        nrm = jnp.sqrt(jnp.sum(tr * tr + tm * tm))
        nrm = jnp.maximum(nrm, 1e-8)
        sre_ref[...] = jnp.broadcast_to(tr / nrm, (_B, _DIM))
        sim_ref[...] = jnp.broadcast_to(tm / nrm, (_B, _DIM))
        ti_ref[...] = jnp.zeros((_B, _TSLOTS), jnp.int32)
        tj_ref[...] = jnp.zeros((_B, _TSLOTS), jnp.int32)
        tmre_ref[...] = jnp.zeros((_B, _TSLOTS), jnp.float32)
        tmim_ref[...] = jnp.zeros((_B, _TSLOTS), jnp.float32)
        tcnt_ref[...] = jnp.zeros((_B, _TSLOTS), jnp.int32)

    eta = jnp.abs(eta_ref[0, 0])
    s_re = sre_ref[...]
    s_im = sim_ref[...]
    h = x_ref[0, :, :]

    c_re = h * s_re
    c_im = h * s_im
    mag2 = c_re * c_re + c_im * c_im

    iota_df = jax.lax.broadcasted_iota(jnp.int32, (_B, _DIM),
                                       1).astype(jnp.float32)

    work = mag2
    top_idxf = []
    for _ in range(_TOPK):
        m = jnp.max(work, axis=1, keepdims=True)
        idxf = jnp.min(jnp.where(work == m, iota_df, float(_DIM)), axis=1,
                       keepdims=True)
        oh = iota_df == idxf
        top_idxf.append(idxf)
        work = jnp.where(oh, -1.0, work)
    top_idx = [v.astype(jnp.int32) for v in top_idxf]

    top_cre = []
    top_cim = []
    top_sre = []
    top_sim = []
    for k in range(_TOPK):
        ohf = (iota_df == top_idxf[k]).astype(jnp.float32)
        top_cre.append(jnp.sum(ohf * c_re, axis=1, keepdims=True))
        top_cim.append(jnp.sum(ohf * c_im, axis=1, keepdims=True))
        top_sre.append(jnp.sum(ohf * s_re, axis=1, keepdims=True))
        top_sim.append(jnp.sum(ohf * s_im, axis=1, keepdims=True))

    score_cols = []
    st_re_cols = []
    st_im_cols = []
    for (i, j) in _PAIRS:
        score_cols.append(top_cre[i] * top_cre[j] + top_cim[i] * top_cim[j])
        pr_re = top_sre[i] * top_sre[j] - top_sim[i] * top_sim[j]
        pr_im = top_sre[i] * top_sim[j] + top_sim[i] * top_sre[j]
        pm = jnp.maximum(jnp.sqrt(pr_re * pr_re + pr_im * pr_im), 1e-8)
        st_re_cols.append(_BETA * pr_re / pm)
        st_im_cols.append(_BETA * pr_im / pm)
    scores = jnp.concatenate(score_cols, axis=1)

    pos = scores > 0.0
    npos = jnp.sum(pos.astype(jnp.int32), axis=1, keepdims=True)
    n_to_bind = (1 + (npos >= 14).astype(jnp.int32)
                 + (npos >= 20).astype(jnp.int32)
                 + (npos >= 27).astype(jnp.int32))
    theta_idx = jnp.minimum(n_to_bind - 1, jnp.maximum(npos - 1, 0))

    iota_pf = jax.lax.broadcasted_iota(jnp.int32, (_B, _NPAIR),
                                       1).astype(jnp.float32)
    work2 = jnp.where(pos, scores, _NEG)
    theta = jnp.full((_B, 1), _NEG, jnp.float32)
    for r in range(4):
        m2 = jnp.max(work2, axis=1, keepdims=True)
        theta = jnp.where(theta_idx == r, m2, theta)
        idx2f = jnp.min(jnp.where(work2 == m2, iota_pf, float(_NPAIR)),
                        axis=1, keepdims=True)
        work2 = jnp.where(iota_pf == idx2f, _NEG, work2)
    selected = (scores >= theta) & (npos > 0)

    ti = ti_ref[...]
    tj = tj_ref[...]
    tm_re = tmre_ref[...]
    tm_im = tmim_ref[...]
    tcnt = tcnt_ref[...]
    active0 = tcnt > 0
    active0f = active0.astype(jnp.float32)
    inact0f = 1.0 - active0f
    n_active0 = jnp.sum(active0f, axis=1, keepdims=True)

    match_sel = []
    any_match_cols = []
    for p, (i, j) in enumerate(_PAIRS):
        pci = top_idx[i]
        pcj = top_idx[j]
        match = active0 & (((ti == pci) & (tj == pcj))
                           | ((ti == pcj) & (tj == pci)))
        any_match_cols.append(jnp.max(match.astype(jnp.float32), axis=1,
                                      keepdims=True))
        match_sel.append(match & selected[:, p:p + 1])
    ms = match_sel
    while len(ms) > 1:
        nxt = [a | b for a, b in zip(ms[::2], ms[1::2])]
        if len(ms) % 2:
            nxt.append(ms[-1])
        ms = nxt
    tcnt = jnp.where(ms[0], _LIFE, tcnt)

    any_match_f = jnp.concatenate(any_match_cols, axis=1)
    app_flag = selected & (any_match_f == 0.0)
    app_flagf = app_flag.astype(jnp.float32)
    lt_p = (jax.lax.broadcasted_iota(jnp.int32, (_NPAIR, _NPAIR), 0)
            < jax.lax.broadcasted_iota(jnp.int32, (_NPAIR, _NPAIR), 1)
            ).astype(jnp.float32)
    rank = jax.lax.dot(app_flagf, lt_p,
                       preferred_element_type=jnp.float32)
    can_append = app_flag & (n_active0 + rank < float(_MAXT))
    lt_s = (jax.lax.broadcasted_iota(jnp.int32, (_TSLOTS, _TSLOTS), 0)
            < jax.lax.broadcasted_iota(jnp.int32, (_TSLOTS, _TSLOTS), 1)
            ).astype(jnp.float32)
    freerank = jax.lax.dot(inact0f, lt_s,
                           preferred_element_type=jnp.float32)
    inact0 = jnp.logical_not(active0)
    for p, (i, j) in enumerate(_PAIRS):
        app = (can_append[:, p:p + 1] & inact0
               & (freerank == rank[:, p:p + 1]))
        ti = jnp.where(app, top_idx[i], ti)
        tj = jnp.where(app, top_idx[j], tj)
        tm_re = jnp.where(app, st_re_cols[p], tm_re)
        tm_im = jnp.where(app, st_im_cols[p], tm_im)
        tcnt = jnp.where(app, _LIFE, tcnt)

    tm_re = tm_re * _GAMMA
    tm_im = tm_im * _GAMMA
    tcnt = tcnt - 1
    tmag = jnp.sqrt(tm_re * tm_re + tm_im * tm_im)
    survive = (tcnt > 0) & (tmag > 1e-6)
    tcnt = jnp.where(survive, tcnt, 0)

    sre_next = s_re
    sim_next = s_im
    ti_ref[...] = ti
    tj_ref[...] = tj
    tmre_ref[...] = tm_re
    tmim_ref[...] = tm_im
    tcnt_ref[...] = tcnt

    aug_re = jnp.zeros((_B, _DIM), jnp.float32)
    aug_im = jnp.zeros((_B, _DIM), jnp.float32)
    iota_d = jax.lax.broadcasted_iota(jnp.int32, (_B, _DIM), 1)
    for k in range(_MAXT):
        alive = tcnt[:, k:k + 1] > 0
        cre = jnp.where(alive, 0.1 * tm_re[:, k:k + 1], 0.0)
        cim = jnp.where(alive, 0.1 * tm_im[:, k:k + 1], 0.0)
        oh = (iota_d == ti[:, k:k + 1]) | (iota_d == tj[:, k:k + 1])
        ohf = oh.astype(jnp.float32)
        aug_re = aug_re + ohf * cre
        aug_im = aug_im + ohf * cim

    ca_re = h * (sre_next + aug_re)
    ca_im = h * (sim_next + aug_im)
    abs_im = jnp.abs(ca_im)
    res_m = ((ca_re > 1e-6) & (abs_im < ca_re)).astype(jnp.float32)
    tor_m = ((ca_re < -1e-6) | (abs_im >= jnp.abs(ca_re))).astype(jnp.float32)
    nonorth = res_m + tor_m
    tb_re = tbre_ref[0:1, :]
    tb_im = tbim_ref[0:1, :]
    upd_re = eta * (ca_re * nonorth + tor_m * tb_re)
    upd_im = eta * (ca_im * nonorth + tor_m * tb_im)
    sn_re = sre_next + upd_re
    sn_im = sim_next + upd_im
    nrm = jnp.sqrt(jnp.sum(sn_re * sn_re + sn_im * sn_im, axis=1,
                           keepdims=True))
    nrm = jnp.maximum(nrm, 1e-8)
    sn_re = sn_re / nrm
    sn_im = sn_im / nrm
    sre_ref[...] = sn_re
    sim_ref[...] = sn_im
    out_ref[0, :, :] = jnp.sqrt(sn_re * sn_re + sn_im * sn_im)


@functools.partial(jax.jit, static_argnames=("interpret",))
def _run(x, tape_re, tape_im, eta, torque_bias_re, torque_bias_im,
         interpret=False):
    B, T, D = x.shape
    xt = jnp.transpose(x, (1, 0, 2))
    tre = tape_re.reshape(1, D)
    tim = tape_im.reshape(1, D)
    tbre = torque_bias_re.reshape(1, D)
    tbim = torque_bias_im.reshape(1, D)
    eta2 = jnp.asarray(eta, jnp.float32).reshape(1, 1)

    out = pl.pallas_call(
        _step_kernel,
        grid=(T,),
        in_specs=[
            pl.BlockSpec((1, B, D), lambda t: (t, 0, 0)),
            pl.BlockSpec((1, D), lambda t: (0, 0)),
            pl.BlockSpec((1, D), lambda t: (0, 0)),
            pl.BlockSpec((1, 1), lambda t: (0, 0)),
            pl.BlockSpec((1, D), lambda t: (0, 0)),
            pl.BlockSpec((1, D), lambda t: (0, 0)),
        ],
        out_specs=pl.BlockSpec((1, B, D), lambda t: (t, 0, 0)),
        out_shape=jax.ShapeDtypeStruct((T, B, D), jnp.float32),
        scratch_shapes=[
            pltpu.VMEM((_B, _DIM), jnp.float32),
            pltpu.VMEM((_B, _DIM), jnp.float32),
            pltpu.VMEM((_B, _TSLOTS), jnp.int32),
            pltpu.VMEM((_B, _TSLOTS), jnp.int32),
            pltpu.VMEM((_B, _TSLOTS), jnp.float32),
            pltpu.VMEM((_B, _TSLOTS), jnp.float32),
            pltpu.VMEM((_B, _TSLOTS), jnp.int32),
        ],
        interpret=interpret,
    )(xt, tre, tim, eta2, tbre, tbim)
    return jnp.transpose(out, (1, 0, 2))


def kernel(x, tape_re, tape_im, eta, torque_bias_re, torque_bias_im):
    return _run(x, tape_re, tape_im, eta, torque_bias_re, torque_bias_im)


# speedup vs baseline: 72.0891x; 1.0713x over previous
import functools

import jax
import jax.numpy as jnp
from jax.experimental import pallas as pl
from jax.experimental.pallas import tpu as pltpu

_DIM = 2048
_B = 8
_TOPK = 8
_MAXT = 16
_TSLOTS = 128
_BETA = 0.05
_GAMMA = 0.9
_LIFE = 5
_PAIRS = [(i, j) for i in range(_TOPK) for j in range(i + 1, _TOPK)]
_NPAIR = len(_PAIRS)
_NEG = -1.0e30


def _step_kernel(x_ref, tre_ref, tim_ref, eta_ref, tbre_ref, tbim_ref,
                 out_ref,
                 sre_ref, sim_ref, ti_ref, tj_ref, tmre_ref, tmim_ref,
                 tcnt_ref):
    t = pl.program_id(0)

    @pl.when(t == 0)
    def _init():
        tr = tre_ref[0:1, :]
        tm = tim_ref[0:1, :]
        nrm = jnp.sqrt(jnp.sum(tr * tr + tm * tm))
        nrm = jnp.maximum(nrm, 1e-8)
        sre_ref[...] = jnp.broadcast_to(tr / nrm, (_B, _DIM))
        sim_ref[...] = jnp.broadcast_to(tm / nrm, (_B, _DIM))
        ti_ref[...] = jnp.zeros((_B, _TSLOTS), jnp.int32)
        tj_ref[...] = jnp.zeros((_B, _TSLOTS), jnp.int32)
        tmre_ref[...] = jnp.zeros((_B, _TSLOTS), jnp.float32)
        tmim_ref[...] = jnp.zeros((_B, _TSLOTS), jnp.float32)
        tcnt_ref[...] = jnp.zeros((_B, _TSLOTS), jnp.int32)

    eta = jnp.abs(eta_ref[0, 0])
    s_re = sre_ref[...]
    s_im = sim_ref[...]
    h = x_ref[0, :, :]

    c_re = h * s_re
    c_im = h * s_im
    mag2 = c_re * c_re + c_im * c_im

    iota_df = jax.lax.broadcasted_iota(jnp.int32, (_B, _DIM),
                                       1).astype(jnp.float32)

    work = mag2
    top_idxf = []
    for _ in range(_TOPK):
        m = jnp.max(work, axis=1, keepdims=True)
        idxf = jnp.min(jnp.where(work == m, iota_df, float(_DIM)), axis=1,
                       keepdims=True)
        oh = iota_df == idxf
        top_idxf.append(idxf)
        work = jnp.where(oh, -1.0, work)
    top_idx = [v.astype(jnp.int32) for v in top_idxf]

    top_cre = []
    top_cim = []
    top_sre = []
    top_sim = []
    for k in range(_TOPK):
        ohf = (iota_df == top_idxf[k]).astype(jnp.float32)
        top_cre.append(jnp.sum(ohf * c_re, axis=1, keepdims=True))
        top_cim.append(jnp.sum(ohf * c_im, axis=1, keepdims=True))
        top_sre.append(jnp.sum(ohf * s_re, axis=1, keepdims=True))
        top_sim.append(jnp.sum(ohf * s_im, axis=1, keepdims=True))

    score_cols = []
    st_re_cols = []
    st_im_cols = []
    for (i, j) in _PAIRS:
        score_cols.append(top_cre[i] * top_cre[j] + top_cim[i] * top_cim[j])
        pr_re = top_sre[i] * top_sre[j] - top_sim[i] * top_sim[j]
        pr_im = top_sre[i] * top_sim[j] + top_sim[i] * top_sre[j]
        pm = jnp.maximum(jnp.sqrt(pr_re * pr_re + pr_im * pr_im), 1e-8)
        st_re_cols.append(_BETA * pr_re / pm)
        st_im_cols.append(_BETA * pr_im / pm)
    scores = jnp.concatenate(score_cols, axis=1)

    pos = scores > 0.0
    npos = jnp.sum(pos.astype(jnp.int32), axis=1, keepdims=True)
    n_to_bind = (1 + (npos >= 14).astype(jnp.int32)
                 + (npos >= 20).astype(jnp.int32)
                 + (npos >= 27).astype(jnp.int32))
    theta_idx = jnp.minimum(n_to_bind - 1, jnp.maximum(npos - 1, 0))

    work2 = jnp.where(pos, scores, _NEG)
    walk_vals = []
    walk_cnts = []
    for r in range(4):
        m2 = jnp.max(work2, axis=1, keepdims=True)
        eq2 = work2 == m2
        walk_vals.append(m2)
        walk_cnts.append(jnp.sum(eq2.astype(jnp.float32), axis=1,
                                 keepdims=True))
        if r < 3:
            work2 = jnp.where(eq2, _NEG, work2)
    theta_idx_f = theta_idx.astype(jnp.float32)
    cum1 = walk_cnts[0]
    cum2 = cum1 + walk_cnts[1]
    cum3 = cum2 + walk_cnts[2]
    theta = jnp.where(
        theta_idx_f < cum1, walk_vals[0],
        jnp.where(theta_idx_f < cum2, walk_vals[1],
                  jnp.where(theta_idx_f < cum3, walk_vals[2],
                            walk_vals[3])))
    selected = (scores >= theta) & (npos > 0)

    ti = ti_ref[...]
    tj = tj_ref[...]
    tm_re = tmre_ref[...]
    tm_im = tmim_ref[...]
    tcnt = tcnt_ref[...]
    active0 = tcnt > 0
    active0f = active0.astype(jnp.float32)
    inact0f = 1.0 - active0f
    n_active0 = jnp.sum(active0f, axis=1, keepdims=True)

    match_sel = []
    any_match_cols = []
    for p, (i, j) in enumerate(_PAIRS):
        pci = top_idx[i]
        pcj = top_idx[j]
        match = active0 & (((ti == pci) & (tj == pcj))
                           | ((ti == pcj) & (tj == pci)))
        any_match_cols.append(jnp.max(match.astype(jnp.float32), axis=1,
                                      keepdims=True))
        match_sel.append(match & selected[:, p:p + 1])
    ms = match_sel
    while len(ms) > 1:
        nxt = [a | b for a, b in zip(ms[::2], ms[1::2])]
        if len(ms) % 2:
            nxt.append(ms[-1])
        ms = nxt
    tcnt = jnp.where(ms[0], _LIFE, tcnt)

    any_match_f = jnp.concatenate(any_match_cols, axis=1)
    app_flag = selected & (any_match_f == 0.0)
    app_flagf = app_flag.astype(jnp.float32)
    lt_p = (jax.lax.broadcasted_iota(jnp.int32, (_NPAIR, _NPAIR), 0)
            < jax.lax.broadcasted_iota(jnp.int32, (_NPAIR, _NPAIR), 1)
            ).astype(jnp.float32)
    rank = jax.lax.dot(app_flagf, lt_p,
                       preferred_element_type=jnp.float32)
    can_append = app_flag & (n_active0 + rank < float(_MAXT))
    lt_s = (jax.lax.broadcasted_iota(jnp.int32, (_TSLOTS, _TSLOTS), 0)
            < jax.lax.broadcasted_iota(jnp.int32, (_TSLOTS, _TSLOTS), 1)
            ).astype(jnp.float32)
    freerank = jax.lax.dot(inact0f, lt_s,
                           preferred_element_type=jnp.float32)
    inact0 = jnp.logical_not(active0)
    for p, (i, j) in enumerate(_PAIRS):
        app = (can_append[:, p:p + 1] & inact0
               & (freerank == rank[:, p:p + 1]))
        ti = jnp.where(app, top_idx[i], ti)
        tj = jnp.where(app, top_idx[j], tj)
        tm_re = jnp.where(app, st_re_cols[p], tm_re)
        tm_im = jnp.where(app, st_im_cols[p], tm_im)
        tcnt = jnp.where(app, _LIFE, tcnt)

    tm_re = tm_re * _GAMMA
    tm_im = tm_im * _GAMMA
    tcnt = tcnt - 1
    tmag = jnp.sqrt(tm_re * tm_re + tm_im * tm_im)
    survive = (tcnt > 0) & (tmag > 1e-6)
    tcnt = jnp.where(survive, tcnt, 0)
    tm_re = jnp.where(survive, tm_re, 0.0)
    tm_im = jnp.where(survive, tm_im, 0.0)

    sre_next = s_re
    sim_next = s_im
    ti_ref[...] = ti
    tj_ref[...] = tj
    tmre_ref[...] = tm_re
    tmim_ref[...] = tm_im
    tcnt_ref[...] = tcnt

    aug_re = jnp.zeros((_B, _DIM), jnp.float32)
    aug_im = jnp.zeros((_B, _DIM), jnp.float32)
    iota_d = jax.lax.broadcasted_iota(jnp.int32, (_B, _DIM), 1)
    for k in range(_MAXT):
        cre = 0.1 * tm_re[:, k:k + 1]
        cim = 0.1 * tm_im[:, k:k + 1]
        oh = (iota_d == ti[:, k:k + 1]) | (iota_d == tj[:, k:k + 1])
        ohf = oh.astype(jnp.float32)
        aug_re = aug_re + ohf * cre
        aug_im = aug_im + ohf * cim

    ca_re = h * (sre_next + aug_re)
    ca_im = h * (sim_next + aug_im)
    abs_im = jnp.abs(ca_im)
    res_m = ((ca_re > 1e-6) & (abs_im < ca_re)).astype(jnp.float32)
    tor_m = ((ca_re < -1e-6) | (abs_im >= jnp.abs(ca_re))).astype(jnp.float32)
    nonorth = res_m + tor_m
    tb_re = tbre_ref[0:1, :]
    tb_im = tbim_ref[0:1, :]
    upd_re = eta * (ca_re * nonorth + tor_m * tb_re)
    upd_im = eta * (ca_im * nonorth + tor_m * tb_im)
    sn_re = sre_next + upd_re
    sn_im = sim_next + upd_im
    nrm = jnp.sqrt(jnp.sum(sn_re * sn_re + sn_im * sn_im, axis=1,
                           keepdims=True))
    nrm = jnp.maximum(nrm, 1e-8)
    sn_re = sn_re / nrm
    sn_im = sn_im / nrm
    sre_ref[...] = sn_re
    sim_ref[...] = sn_im
    out_ref[0, :, :] = jnp.sqrt(sn_re * sn_re + sn_im * sn_im)


@functools.partial(jax.jit, static_argnames=("interpret",))
def _run(x, tape_re, tape_im, eta, torque_bias_re, torque_bias_im,
         interpret=False):
    B, T, D = x.shape
    xt = jnp.transpose(x, (1, 0, 2))
    tre = tape_re.reshape(1, D)
    tim = tape_im.reshape(1, D)
    tbre = torque_bias_re.reshape(1, D)
    tbim = torque_bias_im.reshape(1, D)
    eta2 = jnp.asarray(eta, jnp.float32).reshape(1, 1)

    out = pl.pallas_call(
        _step_kernel,
        grid=(T,),
        in_specs=[
            pl.BlockSpec((1, B, D), lambda t: (t, 0, 0)),
            pl.BlockSpec((1, D), lambda t: (0, 0)),
            pl.BlockSpec((1, D), lambda t: (0, 0)),
            pl.BlockSpec((1, 1), lambda t: (0, 0)),
            pl.BlockSpec((1, D), lambda t: (0, 0)),
            pl.BlockSpec((1, D), lambda t: (0, 0)),
        ],
        out_specs=pl.BlockSpec((1, B, D), lambda t: (t, 0, 0)),
        out_shape=jax.ShapeDtypeStruct((T, B, D), jnp.float32),
        scratch_shapes=[
            pltpu.VMEM((_B, _DIM), jnp.float32),
            pltpu.VMEM((_B, _DIM), jnp.float32),
            pltpu.VMEM((_B, _TSLOTS), jnp.int32),
            pltpu.VMEM((_B, _TSLOTS), jnp.int32),
            pltpu.VMEM((_B, _TSLOTS), jnp.float32),
            pltpu.VMEM((_B, _TSLOTS), jnp.float32),
            pltpu.VMEM((_B, _TSLOTS), jnp.int32),
        ],
        interpret=interpret,
    )(xt, tre, tim, eta2, tbre, tbim)
    return jnp.transpose(out, (1, 0, 2))


def kernel(x, tape_re, tape_im, eta, torque_bias_re, torque_bias_im):
    return _run(x, tape_re, tape_im, eta, torque_bias_re, torque_bias_im)


# speedup vs baseline: 91.6173x; 1.2709x over previous
import functools

import jax
import jax.numpy as jnp
from jax.experimental import pallas as pl
from jax.experimental.pallas import tpu as pltpu

_DIM = 2048
_B = 8
_TOPK = 8
_MAXT = 16
_TSLOTS = 128
_TB = 4
_BETA = 0.05
_GAMMA = 0.9
_LIFE = 5
_PAIRS = [(i, j) for i in range(_TOPK) for j in range(i + 1, _TOPK)]
_NPAIR = len(_PAIRS)
_NEG = -1.0e30


def _make_step_kernel(tb):
    def _step_kernel(x_ref, tre_ref, tim_ref, eta_ref, tbre_ref, tbim_ref,
                     out_ref,
                     sre_ref, sim_ref, ti_ref, tj_ref, tmre_ref, tmim_ref,
                     tcnt_ref):
        return _step_body(tb, x_ref, tre_ref, tim_ref, eta_ref, tbre_ref,
                          tbim_ref, out_ref, sre_ref, sim_ref, ti_ref,
                          tj_ref, tmre_ref, tmim_ref, tcnt_ref)
    return _step_kernel


def _step_body(tb, x_ref, tre_ref, tim_ref, eta_ref, tbre_ref, tbim_ref,
               out_ref,
               sre_ref, sim_ref, ti_ref, tj_ref, tmre_ref, tmim_ref,
               tcnt_ref):
    t = pl.program_id(0)

    @pl.when(t == 0)
    def _init():
        tr = tre_ref[0:1, :]
        tm = tim_ref[0:1, :]
        nrm = jnp.sqrt(jnp.sum(tr * tr + tm * tm))
        nrm = jnp.maximum(nrm, 1e-8)
        sre_ref[...] = jnp.broadcast_to(tr / nrm, (_B, _DIM))
        sim_ref[...] = jnp.broadcast_to(tm / nrm, (_B, _DIM))
        ti_ref[...] = jnp.zeros((_B, _TSLOTS), jnp.int32)
        tj_ref[...] = jnp.zeros((_B, _TSLOTS), jnp.int32)
        tmre_ref[...] = jnp.zeros((_B, _TSLOTS), jnp.float32)
        tmim_ref[...] = jnp.zeros((_B, _TSLOTS), jnp.float32)
        tcnt_ref[...] = jnp.zeros((_B, _TSLOTS), jnp.int32)

    eta = jnp.abs(eta_ref[0, 0])

    def _one_step(tt):
        s_re = sre_ref[...]
        s_im = sim_ref[...]
        h = x_ref[tt, :, :]

        c_re = h * s_re
        c_im = h * s_im
        mag2 = c_re * c_re + c_im * c_im

        iota_df = jax.lax.broadcasted_iota(jnp.int32, (_B, _DIM),
                                           1).astype(jnp.float32)

        work = mag2
        top_idxf = []
        for _ in range(_TOPK):
            m = jnp.max(work, axis=1, keepdims=True)
            eq = work == m
            idxf = jnp.min(jnp.where(eq, iota_df, float(_DIM)), axis=1,
                           keepdims=True)
            top_idxf.append(idxf)
            work = jnp.where(eq, -1.0, work)
        top_idx = [v.astype(jnp.int32) for v in top_idxf]

        top_cre = []
        top_cim = []
        top_sre = []
        top_sim = []
        for k in range(_TOPK):
            ohf = (iota_df == top_idxf[k]).astype(jnp.float32)
            top_cre.append(jnp.sum(ohf * c_re, axis=1, keepdims=True))
            top_cim.append(jnp.sum(ohf * c_im, axis=1, keepdims=True))
            top_sre.append(jnp.sum(ohf * s_re, axis=1, keepdims=True))
            top_sim.append(jnp.sum(ohf * s_im, axis=1, keepdims=True))

        score_cols = []
        st_re_cols = []
        st_im_cols = []
        for (i, j) in _PAIRS:
            score_cols.append(top_cre[i] * top_cre[j]
                              + top_cim[i] * top_cim[j])
            pr_re = top_sre[i] * top_sre[j] - top_sim[i] * top_sim[j]
            pr_im = top_sre[i] * top_sim[j] + top_sim[i] * top_sre[j]
            pm = jnp.maximum(jnp.sqrt(pr_re * pr_re + pr_im * pr_im), 1e-8)
            st_re_cols.append(_BETA * pr_re / pm)
            st_im_cols.append(_BETA * pr_im / pm)
        scores = jnp.concatenate(score_cols, axis=1)

        pos = scores > 0.0
        npos = jnp.sum(pos.astype(jnp.int32), axis=1, keepdims=True)
        n_to_bind = (1 + (npos >= 14).astype(jnp.int32)
                     + (npos >= 20).astype(jnp.int32)
                     + (npos >= 27).astype(jnp.int32))
        theta_idx = jnp.minimum(n_to_bind - 1, jnp.maximum(npos - 1, 0))

        work2 = jnp.where(pos, scores, _NEG)
        walk_vals = []
        walk_cnts = []
        for r in range(4):
            m2 = jnp.max(work2, axis=1, keepdims=True)
            eq2 = work2 == m2
            walk_vals.append(m2)
            walk_cnts.append(jnp.sum(eq2.astype(jnp.float32), axis=1,
                                     keepdims=True))
            if r < 3:
                work2 = jnp.where(eq2, _NEG, work2)
        theta_idx_f = theta_idx.astype(jnp.float32)
        cum1 = walk_cnts[0]
        cum2 = cum1 + walk_cnts[1]
        cum3 = cum2 + walk_cnts[2]
        theta = jnp.where(
            theta_idx_f < cum1, walk_vals[0],
            jnp.where(theta_idx_f < cum2, walk_vals[1],
                      jnp.where(theta_idx_f < cum3, walk_vals[2],
                                walk_vals[3])))
        selected = (scores >= theta) & (npos > 0)

        ti = ti_ref[...]
        tj = tj_ref[...]
        tm_re = tmre_ref[...]
        tm_im = tmim_ref[...]
        tcnt = tcnt_ref[...]
        active0 = tcnt > 0
        active0f = active0.astype(jnp.float32)
        inact0f = 1.0 - active0f
        n_active0 = jnp.sum(active0f, axis=1, keepdims=True)

        match_sel = []
        any_match_cols = []
        for p, (i, j) in enumerate(_PAIRS):
            pci = top_idx[i]
            pcj = top_idx[j]
            match = active0 & (((ti == pci) & (tj == pcj))
                               | ((ti == pcj) & (tj == pci)))
            any_match_cols.append(jnp.max(match.astype(jnp.float32),
                                          axis=1, keepdims=True))
            match_sel.append(match & selected[:, p:p + 1])
        ms = match_sel
        while len(ms) > 1:
            nxt = [a | b for a, b in zip(ms[::2], ms[1::2])]
            if len(ms) % 2:
                nxt.append(ms[-1])
            ms = nxt
        tcnt = jnp.where(ms[0], _LIFE, tcnt)

        any_match_f = jnp.concatenate(any_match_cols, axis=1)
        app_flag = selected & (any_match_f == 0.0)
        app_flagf = app_flag.astype(jnp.float32)
        lt_p = (jax.lax.broadcasted_iota(jnp.int32, (_NPAIR, _NPAIR), 0)
                < jax.lax.broadcasted_iota(jnp.int32, (_NPAIR, _NPAIR), 1)
                ).astype(jnp.float32)
        rank = jax.lax.dot(app_flagf, lt_p,
                           preferred_element_type=jnp.float32)
        can_append = app_flag & (n_active0 + rank < float(_MAXT))
        lt_s = (jax.lax.broadcasted_iota(jnp.int32, (_TSLOTS, _TSLOTS), 0)
                < jax.lax.broadcasted_iota(jnp.int32, (_TSLOTS, _TSLOTS), 1)
                ).astype(jnp.float32)
        freerank = jax.lax.dot(inact0f, lt_s,
                               preferred_element_type=jnp.float32)
        inact0 = jnp.logical_not(active0)
        for p, (i, j) in enumerate(_PAIRS):
            app = (can_append[:, p:p + 1] & inact0
                   & (freerank == rank[:, p:p + 1]))
            ti = jnp.where(app, top_idx[i], ti)
            tj = jnp.where(app, top_idx[j], tj)
            tm_re = jnp.where(app, st_re_cols[p], tm_re)
            tm_im = jnp.where(app, st_im_cols[p], tm_im)
            tcnt = jnp.where(app, _LIFE, tcnt)

        tm_re = tm_re * _GAMMA
        tm_im = tm_im * _GAMMA
        tcnt = tcnt - 1
        tmag = jnp.sqrt(tm_re * tm_re + tm_im * tm_im)
        survive = (tcnt > 0) & (tmag > 1e-6)
        tcnt = jnp.where(survive, tcnt, 0)
        tm_re = jnp.where(survive, tm_re, 0.0)
        tm_im = jnp.where(survive, tm_im, 0.0)

        ti_ref[...] = ti
        tj_ref[...] = tj
        tmre_ref[...] = tm_re
        tmim_ref[...] = tm_im
        tcnt_ref[...] = tcnt

        aug_re = jnp.zeros((_B, _DIM), jnp.float32)
        aug_im = jnp.zeros((_B, _DIM), jnp.float32)
        iota_d = jax.lax.broadcasted_iota(jnp.int32, (_B, _DIM), 1)
        for k in range(_MAXT):
            cre = 0.1 * tm_re[:, k:k + 1]
            cim = 0.1 * tm_im[:, k:k + 1]
            oh = (iota_d == ti[:, k:k + 1]) | (iota_d == tj[:, k:k + 1])
            ohf = oh.astype(jnp.float32)
            aug_re = aug_re + ohf * cre
            aug_im = aug_im + ohf * cim

        ca_re = h * (s_re + aug_re)
        ca_im = h * (s_im + aug_im)
        abs_im = jnp.abs(ca_im)
        res_m = ((ca_re > 1e-6) & (abs_im < ca_re)).astype(jnp.float32)
        tor_m = ((ca_re < -1e-6)
                 | (abs_im >= jnp.abs(ca_re))).astype(jnp.float32)
        nonorth = res_m + tor_m
        tb_re = tbre_ref[0:1, :]
        tb_im = tbim_ref[0:1, :]
        upd_re = eta * (ca_re * nonorth + tor_m * tb_re)
        upd_im = eta * (ca_im * nonorth + tor_m * tb_im)
        sn_re = s_re + upd_re
        sn_im = s_im + upd_im
        nrm = jnp.sqrt(jnp.sum(sn_re * sn_re + sn_im * sn_im, axis=1,
                               keepdims=True))
        nrm = jnp.maximum(nrm, 1e-8)
        sn_re = sn_re / nrm
        sn_im = sn_im / nrm
        sre_ref[...] = sn_re
        sim_ref[...] = sn_im
        out_ref[tt, :, :] = jnp.sqrt(sn_re * sn_re + sn_im * sn_im)

    for tt in range(tb):
        _one_step(tt)


@functools.partial(jax.jit, static_argnames=("interpret",))
def _run(x, tape_re, tape_im, eta, torque_bias_re, torque_bias_im,
         interpret=False):
    B, T, D = x.shape
    xt = jnp.transpose(x, (1, 0, 2))
    tre = tape_re.reshape(1, D)
    tim = tape_im.reshape(1, D)
    tbre = torque_bias_re.reshape(1, D)
    tbim = torque_bias_im.reshape(1, D)
    eta2 = jnp.asarray(eta, jnp.float32).reshape(1, 1)
    tb = _TB if T % _TB == 0 else 1

    out = pl.pallas_call(
        _make_step_kernel(tb),
        grid=(T // tb,),
        in_specs=[
            pl.BlockSpec((tb, B, D), lambda t: (t, 0, 0)),
            pl.BlockSpec((1, D), lambda t: (0, 0)),
            pl.BlockSpec((1, D), lambda t: (0, 0)),
            pl.BlockSpec((1, 1), lambda t: (0, 0)),
            pl.BlockSpec((1, D), lambda t: (0, 0)),
            pl.BlockSpec((1, D), lambda t: (0, 0)),
        ],
        out_specs=pl.BlockSpec((tb, B, D), lambda t: (t, 0, 0)),
        out_shape=jax.ShapeDtypeStruct((T, B, D), jnp.float32),
        scratch_shapes=[
            pltpu.VMEM((_B, _DIM), jnp.float32),
            pltpu.VMEM((_B, _DIM), jnp.float32),
            pltpu.VMEM((_B, _TSLOTS), jnp.int32),
            pltpu.VMEM((_B, _TSLOTS), jnp.int32),
            pltpu.VMEM((_B, _TSLOTS), jnp.float32),
            pltpu.VMEM((_B, _TSLOTS), jnp.float32),
            pltpu.VMEM((_B, _TSLOTS), jnp.int32),
        ],
        interpret=interpret,
    )(xt, tre, tim, eta2, tbre, tbim)
    return jnp.transpose(out, (1, 0, 2))


def kernel(x, tape_re, tape_im, eta, torque_bias_re, torque_bias_im):
    return _run(x, tape_re, tape_im, eta, torque_bias_re, torque_bias_im)


# speedup vs baseline: 97.8803x; 1.0684x over previous
import functools

import jax
import jax.numpy as jnp
from jax.experimental import pallas as pl
from jax.experimental.pallas import tpu as pltpu

_DIM = 2048
_B = 8
_TOPK = 8
_MAXT = 16
_TSLOTS = 128
_TB = 8
_BETA = 0.05
_GAMMA = 0.9
_LIFE = 5
_PAIRS = [(i, j) for i in range(_TOPK) for j in range(i + 1, _TOPK)]
_NPAIR = len(_PAIRS)
_NEG = -1.0e30


def _make_step_kernel(tb):
    def _step_kernel(x_ref, tre_ref, tim_ref, eta_ref, tbre_ref, tbim_ref,
                     out_ref,
                     sre_ref, sim_ref, ti_ref, tj_ref, tmre_ref, tmim_ref,
                     tcnt_ref):
        return _step_body(tb, x_ref, tre_ref, tim_ref, eta_ref, tbre_ref,
                          tbim_ref, out_ref, sre_ref, sim_ref, ti_ref,
                          tj_ref, tmre_ref, tmim_ref, tcnt_ref)
    return _step_kernel


def _step_body(tb, x_ref, tre_ref, tim_ref, eta_ref, tbre_ref, tbim_ref,
               out_ref,
               sre_ref, sim_ref, ti_ref, tj_ref, tmre_ref, tmim_ref,
               tcnt_ref):
    t = pl.program_id(0)

    @pl.when(t == 0)
    def _init():
        tr = tre_ref[0:1, :]
        tm = tim_ref[0:1, :]
        nrm = jnp.sqrt(jnp.sum(tr * tr + tm * tm))
        nrm = jnp.maximum(nrm, 1e-8)
        sre_ref[...] = jnp.broadcast_to(tr / nrm, (_B, _DIM))
        sim_ref[...] = jnp.broadcast_to(tm / nrm, (_B, _DIM))
        ti_ref[...] = jnp.zeros((_B, _TSLOTS), jnp.int32)
        tj_ref[...] = jnp.zeros((_B, _TSLOTS), jnp.int32)
        tmre_ref[...] = jnp.zeros((_B, _TSLOTS), jnp.float32)
        tmim_ref[...] = jnp.zeros((_B, _TSLOTS), jnp.float32)
        tcnt_ref[...] = jnp.zeros((_B, _TSLOTS), jnp.int32)

    eta = jnp.abs(eta_ref[0, 0])

    def _one_step(tt):
        s_re = sre_ref[...]
        s_im = sim_ref[...]
        h = x_ref[tt, :, :]

        c_re = h * s_re
        c_im = h * s_im
        mag2 = c_re * c_re + c_im * c_im

        iota_df = jax.lax.broadcasted_iota(jnp.int32, (_B, _DIM),
                                           1).astype(jnp.float32)

        work = mag2
        top_idxf = []
        for _ in range(_TOPK):
            m = jnp.max(work, axis=1, keepdims=True)
            eq = work == m
            idxf = jnp.min(jnp.where(eq, iota_df, float(_DIM)), axis=1,
                           keepdims=True)
            top_idxf.append(idxf)
            work = jnp.where(eq, -1.0, work)
        top_idx = [v.astype(jnp.int32) for v in top_idxf]

        top_cre = []
        top_cim = []
        top_sre = []
        top_sim = []
        for k in range(_TOPK):
            ohf = (iota_df == top_idxf[k]).astype(jnp.float32)
            top_cre.append(jnp.sum(ohf * c_re, axis=1, keepdims=True))
            top_cim.append(jnp.sum(ohf * c_im, axis=1, keepdims=True))
            top_sre.append(jnp.sum(ohf * s_re, axis=1, keepdims=True))
            top_sim.append(jnp.sum(ohf * s_im, axis=1, keepdims=True))

        score_cols = []
        st_re_cols = []
        st_im_cols = []
        for (i, j) in _PAIRS:
            score_cols.append(top_cre[i] * top_cre[j]
                              + top_cim[i] * top_cim[j])
            pr_re = top_sre[i] * top_sre[j] - top_sim[i] * top_sim[j]
            pr_im = top_sre[i] * top_sim[j] + top_sim[i] * top_sre[j]
            pm = jnp.maximum(jnp.sqrt(pr_re * pr_re + pr_im * pr_im), 1e-8)
            st_re_cols.append(_BETA * pr_re / pm)
            st_im_cols.append(_BETA * pr_im / pm)
        scores = jnp.concatenate(score_cols, axis=1)

        pos = scores > 0.0
        npos = jnp.sum(pos.astype(jnp.int32), axis=1, keepdims=True)
        n_to_bind = (1 + (npos >= 14).astype(jnp.int32)
                     + (npos >= 20).astype(jnp.int32)
                     + (npos >= 27).astype(jnp.int32))
        theta_idx = jnp.minimum(n_to_bind - 1, jnp.maximum(npos - 1, 0))

        theta_idx_f = theta_idx.astype(jnp.float32)
        ngt_cols = []
        for r in range(1, _NPAIR):
            rolled = jnp.concatenate([scores[:, r:], scores[:, :r]],
                                     axis=1)
            ngt_cols.append((rolled > scores).astype(jnp.float32))
        while len(ngt_cols) > 1:
            nxt = [a + b for a, b in zip(ngt_cols[::2], ngt_cols[1::2])]
            if len(ngt_cols) % 2:
                nxt.append(ngt_cols[-1])
            ngt_cols = nxt
        selected = pos & (ngt_cols[0] <= theta_idx_f)

        ti = ti_ref[...]
        tj = tj_ref[...]
        tm_re = tmre_ref[...]
        tm_im = tmim_ref[...]
        tcnt = tcnt_ref[...]
        active0 = tcnt > 0
        active0f = active0.astype(jnp.float32)
        inact0f = 1.0 - active0f
        n_active0 = jnp.sum(active0f, axis=1, keepdims=True)

        match_sel = []
        any_match_cols = []
        for p, (i, j) in enumerate(_PAIRS):
            pci = top_idx[i]
            pcj = top_idx[j]
            match = active0 & (((ti == pci) & (tj == pcj))
                               | ((ti == pcj) & (tj == pci)))
            any_match_cols.append(jnp.max(match.astype(jnp.float32),
                                          axis=1, keepdims=True))
            match_sel.append(match & selected[:, p:p + 1])
        ms = match_sel
        while len(ms) > 1:
            nxt = [a | b for a, b in zip(ms[::2], ms[1::2])]
            if len(ms) % 2:
                nxt.append(ms[-1])
            ms = nxt
        tcnt = jnp.where(ms[0], _LIFE, tcnt)

        any_match_f = jnp.concatenate(any_match_cols, axis=1)
        app_flag = selected & (any_match_f == 0.0)
        app_flagf = app_flag.astype(jnp.float32)
        lt_p = (jax.lax.broadcasted_iota(jnp.int32, (_NPAIR, _NPAIR), 0)
                < jax.lax.broadcasted_iota(jnp.int32, (_NPAIR, _NPAIR), 1)
                ).astype(jnp.float32)
        rank = jax.lax.dot(app_flagf, lt_p,
                           preferred_element_type=jnp.float32)
        can_append = app_flag & (n_active0 + rank < float(_MAXT))
        lt_s = (jax.lax.broadcasted_iota(jnp.int32, (_TSLOTS, _TSLOTS), 0)
                < jax.lax.broadcasted_iota(jnp.int32, (_TSLOTS, _TSLOTS), 1)
                ).astype(jnp.float32)
        freerank = jax.lax.dot(inact0f, lt_s,
                               preferred_element_type=jnp.float32)
        inact0 = jnp.logical_not(active0)
        for p, (i, j) in enumerate(_PAIRS):
            app = (can_append[:, p:p + 1] & inact0
                   & (freerank == rank[:, p:p + 1]))
            ti = jnp.where(app, top_idx[i], ti)
            tj = jnp.where(app, top_idx[j], tj)
            tm_re = jnp.where(app, st_re_cols[p], tm_re)
            tm_im = jnp.where(app, st_im_cols[p], tm_im)
            tcnt = jnp.where(app, _LIFE, tcnt)

        tm_re = tm_re * _GAMMA
        tm_im = tm_im * _GAMMA
        tcnt = tcnt - 1
        tmag = jnp.sqrt(tm_re * tm_re + tm_im * tm_im)
        survive = (tcnt > 0) & (tmag > 1e-6)
        tcnt = jnp.where(survive, tcnt, 0)
        tm_re = jnp.where(survive, tm_re, 0.0)
        tm_im = jnp.where(survive, tm_im, 0.0)

        ti_ref[...] = ti
        tj_ref[...] = tj
        tmre_ref[...] = tm_re
        tmim_ref[...] = tm_im
        tcnt_ref[...] = tcnt

        aug_re = jnp.zeros((_B, _DIM), jnp.float32)
        aug_im = jnp.zeros((_B, _DIM), jnp.float32)
        iota_d = jax.lax.broadcasted_iota(jnp.int32, (_B, _DIM), 1)
        for k in range(_MAXT):
            cre = 0.1 * tm_re[:, k:k + 1]
            cim = 0.1 * tm_im[:, k:k + 1]
            oh = (iota_d == ti[:, k:k + 1]) | (iota_d == tj[:, k:k + 1])
            ohf = oh.astype(jnp.float32)
            aug_re = aug_re + ohf * cre
            aug_im = aug_im + ohf * cim

        ca_re = h * (s_re + aug_re)
        ca_im = h * (s_im + aug_im)
        abs_im = jnp.abs(ca_im)
        res_m = ((ca_re > 1e-6) & (abs_im < ca_re)).astype(jnp.float32)
        tor_m = ((ca_re < -1e-6)
                 | (abs_im >= jnp.abs(ca_re))).astype(jnp.float32)
        nonorth = res_m + tor_m
        tb_re = tbre_ref[0:1, :]
        tb_im = tbim_ref[0:1, :]
        upd_re = eta * (ca_re * nonorth + tor_m * tb_re)
        upd_im = eta * (ca_im * nonorth + tor_m * tb_im)
        sn_re = s_re + upd_re
        sn_im = s_im + upd_im
        nrm = jnp.sqrt(jnp.sum(sn_re * sn_re + sn_im * sn_im, axis=1,
                               keepdims=True))
        nrm = jnp.maximum(nrm, 1e-8)
        sn_re = sn_re / nrm
        sn_im = sn_im / nrm
        sre_ref[...] = sn_re
        sim_ref[...] = sn_im
        out_ref[tt, :, :] = jnp.sqrt(sn_re * sn_re + sn_im * sn_im)

    for tt in range(tb):
        _one_step(tt)


@functools.partial(jax.jit, static_argnames=("interpret",))
def _run(x, tape_re, tape_im, eta, torque_bias_re, torque_bias_im,
         interpret=False):
    B, T, D = x.shape
    xt = jnp.transpose(x, (1, 0, 2))
    tre = tape_re.reshape(1, D)
    tim = tape_im.reshape(1, D)
    tbre = torque_bias_re.reshape(1, D)
    tbim = torque_bias_im.reshape(1, D)
    eta2 = jnp.asarray(eta, jnp.float32).reshape(1, 1)
    tb = _TB if T % _TB == 0 else 1

    out = pl.pallas_call(
        _make_step_kernel(tb),
        grid=(T // tb,),
        in_specs=[
            pl.BlockSpec((tb, B, D), lambda t: (t, 0, 0)),
            pl.BlockSpec((1, D), lambda t: (0, 0)),
            pl.BlockSpec((1, D), lambda t: (0, 0)),
            pl.BlockSpec((1, 1), lambda t: (0, 0)),
            pl.BlockSpec((1, D), lambda t: (0, 0)),
            pl.BlockSpec((1, D), lambda t: (0, 0)),
        ],
        out_specs=pl.BlockSpec((tb, B, D), lambda t: (t, 0, 0)),
        out_shape=jax.ShapeDtypeStruct((T, B, D), jnp.float32),
        scratch_shapes=[
            pltpu.VMEM((_B, _DIM), jnp.float32),
            pltpu.VMEM((_B, _DIM), jnp.float32),
            pltpu.VMEM((_B, _TSLOTS), jnp.int32),
            pltpu.VMEM((_B, _TSLOTS), jnp.int32),
            pltpu.VMEM((_B, _TSLOTS), jnp.float32),
            pltpu.VMEM((_B, _TSLOTS), jnp.float32),
            pltpu.VMEM((_B, _TSLOTS), jnp.int32),
        ],
        interpret=interpret,
    )(xt, tre, tim, eta2, tbre, tbim)
    return jnp.transpose(out, (1, 0, 2))


def kernel(x, tape_re, tape_im, eta, torque_bias_re, torque_bias_im):
    return _run(x, tape_re, tape_im, eta, torque_bias_re, torque_bias_im)
